# Initial kernel scaffold; baseline (speedup 1.0000x reference)
import functools

import jax
import jax.numpy as jnp
from jax import lax
from jax.experimental import pallas as pl
from jax.experimental.pallas import tpu as pltpu
from jax.experimental.pallas import tpu_sc as plsc

NC = 2
NS = 16
LANES = 16
CHUNK = 80
ZROWS = 125


def _lane_broadcast(vec, i):
    idx = jnp.full((LANES,), i, jnp.int32)
    return jnp.take(vec, idx, mode=lax.GatherScatterMode.PROMISE_IN_BOUNDS)


def _make_sc_spmm(n_nodes, n_edges, d_feat):
    nw = NC * NS
    epw = n_edges // nw
    nchunk = epw // CHUNK
    rows_per_sub = n_nodes // NS

    mesh = plsc.VectorSubcoreMesh(core_axis_name="c", subcore_axis_name="s")

    @functools.partial(
        pl.kernel,
        out_type=jax.ShapeDtypeStruct((NC, n_nodes, d_feat), jnp.float32),
        mesh=mesh,
        scratch_types=[
            pltpu.VMEM((CHUNK,), jnp.int32),
            pltpu.VMEM((CHUNK,), jnp.int32),
            pltpu.VMEM((CHUNK,), jnp.float32),
            pltpu.VMEM((CHUNK, d_feat), jnp.float32),
            pltpu.VMEM((ZROWS, d_feat), jnp.float32),
            pltpu.VMEM_SHARED((n_nodes, d_feat), jnp.float32),
            pltpu.SemaphoreType.DMA,
        ],
    )
    def spmm(emb_hbm, col_hbm, row_hbm, val_hbm, out_hbm,
             col_v, row_v, val_v, rows_v, zbuf, acc_sh, sem):
        c = lax.axis_index("c")
        s = lax.axis_index("s")

        zero16 = jnp.zeros((LANES,), jnp.float32)

        @pl.loop(0, ZROWS)
        def _(i):
            for j in range(d_feat // LANES):
                zbuf[i, pl.ds(j * LANES, LANES)] = zero16

        rbase = s * rows_per_sub
        for k in range(rows_per_sub // ZROWS):
            pltpu.sync_copy(zbuf, acc_sh.at[pl.ds(rbase + k * ZROWS, ZROWS)])
        plsc.subcore_barrier()

        ebase = (c * NS + s) * epw

        @pl.loop(0, nchunk)
        def _(g):
            off = ebase + g * CHUNK
            pltpu.sync_copy(col_hbm.at[pl.ds(off, CHUNK)], col_v)
            pltpu.sync_copy(row_hbm.at[pl.ds(off, CHUNK)], row_v)
            pltpu.sync_copy(val_hbm.at[pl.ds(off, CHUNK)], val_v)
            pltpu.async_copy(emb_hbm.at[col_v], rows_v, sem).wait()

            @pl.loop(0, CHUNK // LANES)
            def _(g16):
                vv = val_v[pl.ds(g16 * LANES, LANES)]

                @pl.loop(0, LANES)
                def _(i):
                    vb = _lane_broadcast(vv, i)
                    e = g16 * LANES + i
                    for j in range(d_feat // LANES):
                        sl = pl.ds(j * LANES, LANES)
                        rows_v[e, sl] = rows_v[e, sl] * vb

            pltpu.sync_copy(rows_v, acc_sh.at[row_v], add=True)

        plsc.subcore_barrier()
        pltpu.sync_copy(
            acc_sh.at[pl.ds(rbase, rows_per_sub)],
            out_hbm.at[c, pl.ds(rbase, rows_per_sub)],
        )

    return spmm


def _add_partials_body(a_ref, o_ref):
    o_ref[...] = a_ref[0] + a_ref[1]


def kernel(embeds, edge_index, adj_values):
    n_nodes, d_feat = embeds.shape
    edge_index = edge_index.astype(jnp.int32)
    row = edge_index[0]
    col = edge_index[1]
    n_edges = adj_values.shape[0]

    spmm = _make_sc_spmm(n_nodes, n_edges, d_feat)
    partials = spmm(embeds, col, row, adj_values)

    out = pl.pallas_call(
        _add_partials_body,
        out_shape=jax.ShapeDtypeStruct((n_nodes, d_feat), jnp.float32),
    )(partials)
    return out


# speedup vs baseline: 4.5749x; 4.5749x over previous
import functools

import jax
import jax.numpy as jnp
from jax import lax
from jax.experimental import pallas as pl
from jax.experimental.pallas import tpu as pltpu
from jax.experimental.pallas import tpu_sc as plsc

NC = 2
NS = 16
LANES = 16
CHUNK = 80
ZROWS = 200


def _lane_broadcast(vec, i):
    idx = jnp.full((LANES,), i, jnp.int32)
    dnums = lax.GatherDimensionNumbers(
        offset_dims=(), collapsed_slice_dims=(0,), start_index_map=(0,))
    return lax.gather(vec, idx[:, None], dnums, (1,),
                      mode=lax.GatherScatterMode.PROMISE_IN_BOUNDS)


def _make_sc_spmm(n_nodes, n_edges, d_feat):
    nw = NC * NS
    epw = n_edges // nw
    nchunk = epw // CHUNK
    nslice = n_nodes // ZROWS
    slices_per_sub = (nslice + NS - 1) // NS

    mesh = plsc.VectorSubcoreMesh(core_axis_name="c", subcore_axis_name="s")

    @functools.partial(
        pl.kernel,
        out_type=jax.ShapeDtypeStruct((NC, n_nodes, d_feat), jnp.float32),
        mesh=mesh,
        scratch_types=[
            pltpu.VMEM((CHUNK,), jnp.int32),
            pltpu.VMEM((CHUNK,), jnp.int32),
            pltpu.VMEM((CHUNK,), jnp.float32),
            pltpu.VMEM((CHUNK, d_feat), jnp.float32),
            pltpu.VMEM((ZROWS, d_feat), jnp.float32),
            pltpu.VMEM_SHARED((n_nodes, d_feat), jnp.float32),
            pltpu.SemaphoreType.DMA,
        ],
    )
    def spmm(emb_hbm, col_hbm, row_hbm, val_hbm, out_hbm,
             col_v, row_v, val_v, rows_v, zbuf, acc_sh, sem):
        c = lax.axis_index("c")
        s = lax.axis_index("s")

        zero16 = jnp.zeros((LANES,), jnp.float32)

        @pl.loop(0, ZROWS)
        def _(i):
            for j in range(d_feat // LANES):
                zbuf[i, pl.ds(j * LANES, LANES)] = zero16

        for k in range(slices_per_sub):
            sl_idx = s + NS * k

            @pl.when(sl_idx < nslice)
            def _():
                off = pl.multiple_of(sl_idx * ZROWS, 8)
                pltpu.sync_copy(zbuf, acc_sh.at[pl.ds(off, ZROWS)])
        plsc.subcore_barrier()

        ebase = (c * NS + s) * epw

        @pl.loop(0, nchunk)
        def _(g):
            off = ebase + g * CHUNK
            pltpu.sync_copy(col_hbm.at[pl.ds(off, CHUNK)], col_v)
            pltpu.sync_copy(row_hbm.at[pl.ds(off, CHUNK)], row_v)
            pltpu.sync_copy(val_hbm.at[pl.ds(off, CHUNK)], val_v)
            pltpu.async_copy(emb_hbm.at[col_v], rows_v, sem).wait()

            @pl.loop(0, CHUNK // LANES)
            def _(g16):
                vv = val_v[pl.ds(g16 * LANES, LANES)]

                @pl.loop(0, LANES)
                def _(i):
                    vb = _lane_broadcast(vv, i)
                    e = g16 * LANES + i
                    for j in range(d_feat // LANES):
                        sl = pl.ds(j * LANES, LANES)
                        rows_v[e, sl] = rows_v[e, sl] * vb

            pltpu.sync_copy(rows_v, acc_sh.at[row_v], add=True)

        plsc.subcore_barrier()
        for k in range(slices_per_sub):
            sl_idx = s + NS * k

            @pl.when(sl_idx < nslice)
            def _():
                off = pl.multiple_of(sl_idx * ZROWS, 8)
                pltpu.sync_copy(
                    acc_sh.at[pl.ds(off, ZROWS)],
                    out_hbm.at[c, pl.ds(off, ZROWS)],
                )

    return spmm


def _add_partials_body(a_ref, o_ref):
    o_ref[...] = a_ref[0] + a_ref[1]


def kernel(embeds, edge_index, adj_values):
    n_nodes, d_feat = embeds.shape
    edge_index = edge_index.astype(jnp.int32)
    row = edge_index[0]
    col = edge_index[1]
    n_edges = adj_values.shape[0]

    spmm = _make_sc_spmm(n_nodes, n_edges, d_feat)
    partials = spmm(embeds, col, row, adj_values)

    out = pl.pallas_call(
        _add_partials_body,
        out_shape=jax.ShapeDtypeStruct((n_nodes, d_feat), jnp.float32),
    )(partials)
    return out
